# baseline (speedup 1.0000x reference)
import functools

import jax
import jax.numpy as jnp
from jax import lax
from jax.experimental import pallas as pl
from jax.experimental.pallas import tpu as pltpu
from jax.experimental.pallas import tpu_sc as plsc

_B = 64
_NC = 2
_NS = 16
_LANES = 16


def _h_body(x_ref, w1_ref, h_ref):
    h_ref[...] = jnp.dot(x_ref[...], w1_ref[...],
                         preferred_element_type=jnp.float32)


def _h_matmul(x, w1):
    n, _ = x.shape
    h = w1.shape[1]
    return pl.pallas_call(
        _h_body,
        out_shape=jax.ShapeDtypeStruct((n, h), jnp.float32),
    )(x, w1)


_CH = 96


def _make_sc_msg_pass(n, h, e):
    nw = _NC * _NS
    ept = e // nw
    nch = ept // _CH
    tail = ept - nch * _CH
    rp = 1000
    nt = n // rp
    assert ept * nw == e and nch % 2 == 0 and 0 < tail <= _CH
    assert rp % 8 == 0 and rp * nt == n and nt <= _NS and tail % 8 == 0

    mesh = plsc.VectorSubcoreMesh(core_axis_name="c", subcore_axis_name="s")

    main_pt = nch * _CH
    e_main = main_pt * nw

    @functools.partial(
        pl.kernel,
        out_type=jax.ShapeDtypeStruct((_NC, n, h), jnp.float32),
        mesh=mesh,
        scratch_types=[
            pltpu.VMEM((ept,), jnp.int32),
            pltpu.VMEM((nch, _CH), jnp.int32),
            pltpu.VMEM((tail,), jnp.int32),
            pltpu.VMEM((_CH, h), jnp.float32),
            pltpu.VMEM((_CH, h), jnp.float32),
            pltpu.VMEM_SHARED((n, h), jnp.float32),
            pltpu.SemaphoreType.DMA,
            pltpu.SemaphoreType.DMA,
            pltpu.SemaphoreType.DMA,
            pltpu.SemaphoreType.DMA,
        ],
    )
    def sc_msg_pass(h_hbm, src_hbm, dst3_hbm, dstt_hbm, zeros_hbm,
                    out_hbm, src_v, dst_v, dstt_v, rows0, rows1,
                    acc_sh, sem0, sem1, sems0, sems1):
        cid = lax.axis_index("c")
        sid = lax.axis_index("s")
        wid = cid * _NS + sid

        @pl.when(sid < nt)
        def _():
            pltpu.sync_copy(zeros_hbm, acc_sh.at[pl.ds(sid * rp, rp)])

        pltpu.sync_copy(src_hbm.at[pl.ds(wid * main_pt, main_pt)],
                        src_v.at[pl.ds(0, main_pt)])
        pltpu.sync_copy(src_hbm.at[pl.ds(e_main + wid * tail, tail)],
                        src_v.at[pl.ds(main_pt, tail)])
        pltpu.sync_copy(dst3_hbm.at[wid], dst_v)
        pltpu.sync_copy(dstt_hbm.at[wid], dstt_v)
        plsc.subcore_barrier()

        def sidx(c):
            return src_v.at[pl.ds(c * _CH, _CH)]

        def wait_gather(rows, sem):
            pltpu.make_async_copy(h_hbm.at[sidx(0)], rows, sem).wait()

        def wait_scatter(rows, sem):
            pltpu.make_async_copy(rows, acc_sh.at[dst_v.at[0]], sem).wait()

        pltpu.async_copy(h_hbm.at[sidx(0)], rows0, sem0)
        pltpu.async_copy(h_hbm.at[sidx(1)], rows1, sem1)

        def body(g, carry):
            a = 2 * g
            wait_gather(rows0, sem0)
            pltpu.async_copy(rows0, acc_sh.at[dst_v.at[a]], sems0, add=True)
            wait_gather(rows1, sem1)
            pltpu.async_copy(rows1, acc_sh.at[dst_v.at[a + 1]], sems1,
                             add=True)

            @pl.when(a + 2 < nch)
            def _():
                wait_scatter(rows0, sems0)
                pltpu.async_copy(h_hbm.at[sidx(a + 2)], rows0, sem0)

            @pl.when(a + 3 < nch)
            def _():
                wait_scatter(rows1, sems1)
                pltpu.async_copy(h_hbm.at[sidx(a + 3)], rows1, sem1)

            return carry

        lax.fori_loop(0, nch // 2, body, 0)
        wait_scatter(rows0, sems0)
        wait_scatter(rows1, sems1)

        rowst = rows0.at[pl.ds(0, tail)]
        pltpu.async_copy(h_hbm.at[src_v.at[pl.ds(main_pt, tail)]],
                         rowst, sem0).wait()
        pltpu.sync_copy(rowst, acc_sh.at[dstt_v], add=True)
        plsc.subcore_barrier()

        @pl.when(sid < nt)
        def _():
            pltpu.sync_copy(acc_sh.at[pl.ds(sid * rp, rp)],
                            out_hbm.at[cid, pl.ds(sid * rp, rp)])

    return sc_msg_pass


def _s_body(g_ref, w2_ref, wout_ref, s_ref):
    g = jnp.maximum(g_ref[0] + g_ref[1], 0.0)
    v = jnp.dot(w2_ref[...], wout_ref[...],
                preferred_element_type=jnp.float32)
    s_ref[...] = jnp.dot(g, v, preferred_element_type=jnp.float32)[:, 0]


def _s_matvec(g_part, w2, w_out):
    n = g_part.shape[1]
    return pl.pallas_call(
        _s_body,
        out_shape=jax.ShapeDtypeStruct((n,), jnp.float32),
    )(g_part, w2, w_out)


def _make_sc_scalar_pass(n, e):
    nw = _NC * _NS
    ept = e // nw
    nvec = ept // _LANES
    assert ept * nw == e and nvec * _LANES == ept

    mesh = plsc.VectorSubcoreMesh(core_axis_name="c", subcore_axis_name="s")

    @functools.partial(
        pl.kernel,
        out_type=jax.ShapeDtypeStruct((nw, _LANES, _B), jnp.float32),
        mesh=mesh,
        scratch_types=[
            pltpu.VMEM((ept,), jnp.int32),
            pltpu.VMEM((ept,), jnp.int32),
            pltpu.VMEM((n,), jnp.float32),
            pltpu.VMEM((n,), jnp.int32),
            pltpu.VMEM((_LANES, _B), jnp.float32),
        ],
        compiler_params=pltpu.CompilerParams(needs_layout_passes=False),
    )
    def sc_scalar_pass(s_hbm, batch_hbm, src_hbm, dst_hbm, out_hbm,
                       src_v, dst_v, s_v, b_v, acc_v):
        cid = lax.axis_index("c")
        sid = lax.axis_index("s")
        wid = cid * _NS + sid
        pltpu.sync_copy(s_hbm, s_v)
        pltpu.sync_copy(batch_hbm, b_v)
        pltpu.sync_copy(src_hbm.at[pl.ds(wid * ept, ept)], src_v)
        pltpu.sync_copy(dst_hbm.at[pl.ds(wid * ept, ept)], dst_v)

        zero = jnp.zeros((_LANES,), jnp.float32)
        for r in range(_LANES):
            for c0 in range(_B // _LANES):
                acc_v[r, pl.ds(c0 * _LANES, _LANES)] = zero

        lane = lax.iota(jnp.int32, _LANES)

        def body(i, carry):
            sl = pl.ds(i * _LANES, _LANES)
            src16 = src_v[sl]
            dst16 = dst_v[sl]
            sval = plsc.load_gather(s_v, [src16])
            bval = plsc.load_gather(b_v, [dst16])
            plsc.addupdate_scatter(acc_v, [lane, bval], sval)
            return carry

        lax.fori_loop(0, nvec, body, 0)
        pltpu.sync_copy(acc_v, out_hbm.at[wid])

    return sc_scalar_pass


def _out_body(acc_ref, batch_ref, bout_ref, out_ref):
    acc = acc_ref[...]
    num = jnp.sum(acc.reshape(acc.shape[0] * acc.shape[1], _B),
                  axis=0, keepdims=True)
    bt = batch_ref[...]
    iot = lax.broadcasted_iota(jnp.int32, (1, _B), 1)
    cnt = jnp.sum((bt == iot).astype(jnp.float32), axis=0,
                  keepdims=True)
    pooled = num / jnp.maximum(cnt, 1.0)
    out_ref[...] = jax.nn.sigmoid(pooled + bout_ref[0, 0])


def _pool_tail(acc, batch2d, bout2d):
    return pl.pallas_call(
        _out_body,
        out_shape=jax.ShapeDtypeStruct((1, _B), jnp.float32),
    )(acc, batch2d, bout2d)


def kernel(x, edge_index, batch, W1, W2, W_out, b_out):
    n, _ = x.shape
    h_dim = W1.shape[1]
    e = edge_index.shape[1]

    src = edge_index[0].astype(jnp.int32)
    dst = edge_index[1].astype(jnp.int32)
    batch_i = batch.astype(jnp.int32)

    nw = _NC * _NS
    ept = e // nw
    nch = ept // _CH
    e_main = nch * _CH * nw
    dst3 = dst[:e_main].reshape(nw, nch, _CH)
    dstt = dst[e_main:].reshape(nw, -1)

    h = _h_matmul(x, W1)
    zeros = jnp.zeros((1000, h_dim), jnp.float32)
    g_part = _make_sc_msg_pass(n, h_dim, e)(h, src, dst3, dstt, zeros)
    s = _s_matvec(g_part, W2, W_out)
    acc = _make_sc_scalar_pass(n, e)(s, batch_i, src, dst)
    out = _pool_tail(acc, batch_i.reshape(n, 1), b_out.reshape(1, 1))
    return out.reshape(_B, 1)


# speedup vs baseline: 1.3040x; 1.3040x over previous
import functools

import jax
import jax.numpy as jnp
from jax import lax
from jax.experimental import pallas as pl
from jax.experimental.pallas import tpu as pltpu
from jax.experimental.pallas import tpu_sc as plsc

_B = 64
_NC = 2
_NS = 16
_LANES = 16


def _h_body(x_ref, w1_ref, h_ref):
    h_ref[...] = jnp.dot(x_ref[...], w1_ref[...],
                         preferred_element_type=jnp.float32)


def _h_matmul(x, w1):
    n, _ = x.shape
    h = w1.shape[1]
    return pl.pallas_call(
        _h_body,
        out_shape=jax.ShapeDtypeStruct((n, h), jnp.float32),
    )(x, w1)


_CH = 96


def _make_sc_msg_pass(n, h, e):
    nw = _NC * _NS
    ept = e // nw
    nch = ept // _CH
    tail = ept - nch * _CH
    rp = 1000
    nt = n // rp
    assert ept * nw == e and nch % 2 == 0 and 0 < tail <= _CH
    assert rp % 8 == 0 and rp * nt == n and nt <= _NS and tail % 8 == 0

    mesh = plsc.VectorSubcoreMesh(core_axis_name="c", subcore_axis_name="s")

    @functools.partial(
        pl.kernel,
        out_type=jax.ShapeDtypeStruct((_NC, n, h), jnp.float32),
        mesh=mesh,
        scratch_types=[
            pltpu.VMEM((ept,), jnp.int32),
            pltpu.VMEM((nch, _CH), jnp.int32),
            pltpu.VMEM((tail,), jnp.int32),
            pltpu.VMEM((_CH, h), jnp.float32),
            pltpu.VMEM((_CH, h), jnp.float32),
            pltpu.VMEM_SHARED((n, h), jnp.float32),
            pltpu.SemaphoreType.DMA,
            pltpu.SemaphoreType.DMA,
        ],
    )
    def sc_msg_pass(src_hbm, dst3_hbm, dstt_hbm, h_hbm, zeros_hbm, out_hbm,
                    src_v, dst_v, dstt_v, rows0, rows1, acc_sh, sem0, sem1):
        cid = lax.axis_index("c")
        sid = lax.axis_index("s")
        wid = cid * _NS + sid

        @pl.when(sid < nt)
        def _():
            pltpu.sync_copy(zeros_hbm, acc_sh.at[pl.ds(sid * rp, rp)])

        pltpu.sync_copy(src_hbm.at[pl.ds(wid * ept, ept)], src_v)
        pltpu.sync_copy(dst3_hbm.at[wid], dst_v)
        pltpu.sync_copy(dstt_hbm.at[wid], dstt_v)
        plsc.subcore_barrier()

        def sidx(c):
            return src_v.at[pl.ds(c * _CH, _CH)]

        def didx(c):
            return dst_v.at[c]

        pltpu.async_copy(h_hbm.at[sidx(0)], rows0, sem0)
        pltpu.async_copy(h_hbm.at[sidx(1)], rows1, sem1)

        def body(g, carry):
            a = 2 * g
            pltpu.make_async_copy(h_hbm.at[sidx(a)], rows0, sem0).wait()
            pltpu.sync_copy(rows0, acc_sh.at[didx(a)], add=True)

            @pl.when(a + 2 < nch)
            def _():
                pltpu.async_copy(h_hbm.at[sidx(a + 2)], rows0, sem0)

            pltpu.make_async_copy(h_hbm.at[sidx(a + 1)], rows1, sem1).wait()
            pltpu.sync_copy(rows1, acc_sh.at[didx(a + 1)], add=True)

            @pl.when(a + 3 < nch)
            def _():
                pltpu.async_copy(h_hbm.at[sidx(a + 3)], rows1, sem1)

            return carry

        lax.fori_loop(0, nch // 2, body, 0)

        rowst = rows0.at[pl.ds(0, tail)]
        pltpu.async_copy(h_hbm.at[src_v.at[pl.ds(nch * _CH, tail)]],
                         rowst, sem0).wait()
        pltpu.sync_copy(rowst, acc_sh.at[dstt_v], add=True)
        plsc.subcore_barrier()

        @pl.when(sid < nt)
        def _():
            pltpu.sync_copy(acc_sh.at[pl.ds(sid * rp, rp)],
                            out_hbm.at[cid, pl.ds(sid * rp, rp)])

    return sc_msg_pass


def _s_body(g_ref, w2_ref, wout_ref, s_ref):
    g = jnp.maximum(g_ref[0] + g_ref[1], 0.0)
    v = jnp.dot(w2_ref[...], wout_ref[...],
                preferred_element_type=jnp.float32)
    s_ref[...] = jnp.dot(g, v, preferred_element_type=jnp.float32)[:, 0]


def _s_matvec(g_part, w2, w_out):
    n = g_part.shape[1]
    return pl.pallas_call(
        _s_body,
        out_shape=jax.ShapeDtypeStruct((n,), jnp.float32),
    )(g_part, w2, w_out)


def _make_sc_scalar_pass(n, e):
    nw = _NC * _NS
    ept = e // nw
    nvec = ept // _LANES
    assert ept * nw == e and nvec * _LANES == ept

    mesh = plsc.VectorSubcoreMesh(core_axis_name="c", subcore_axis_name="s")

    @functools.partial(
        pl.kernel,
        out_type=jax.ShapeDtypeStruct((nw, _LANES, _B), jnp.float32),
        mesh=mesh,
        scratch_types=[
            pltpu.VMEM((ept,), jnp.int32),
            pltpu.VMEM((ept,), jnp.int32),
            pltpu.VMEM((n,), jnp.float32),
            pltpu.VMEM((n,), jnp.int32),
            pltpu.VMEM((_LANES, _B), jnp.float32),
        ],
        compiler_params=pltpu.CompilerParams(needs_layout_passes=False),
    )
    def sc_scalar_pass(src_hbm, dst_hbm, s_hbm, batch_hbm, out_hbm,
                       src_v, dst_v, s_v, b_v, acc_v):
        cid = lax.axis_index("c")
        sid = lax.axis_index("s")
        wid = cid * _NS + sid
        pltpu.sync_copy(s_hbm, s_v)
        pltpu.sync_copy(batch_hbm, b_v)
        pltpu.sync_copy(src_hbm.at[pl.ds(wid * ept, ept)], src_v)
        pltpu.sync_copy(dst_hbm.at[pl.ds(wid * ept, ept)], dst_v)

        zero = jnp.zeros((_LANES,), jnp.float32)
        for r in range(_LANES):
            for c0 in range(_B // _LANES):
                acc_v[r, pl.ds(c0 * _LANES, _LANES)] = zero

        lane = lax.iota(jnp.int32, _LANES)

        def body(i, carry):
            sl = pl.ds(i * _LANES, _LANES)
            src16 = src_v[sl]
            dst16 = dst_v[sl]
            sval = plsc.load_gather(s_v, [src16])
            bval = plsc.load_gather(b_v, [dst16])
            plsc.addupdate_scatter(acc_v, [lane, bval], sval)
            return carry

        lax.fori_loop(0, nvec, body, 0)
        pltpu.sync_copy(acc_v, out_hbm.at[wid])

    return sc_scalar_pass


def _out_body(acc_ref, batch_ref, bout_ref, out_ref):
    acc = acc_ref[...]
    num = jnp.sum(acc.reshape(acc.shape[0] * acc.shape[1], _B),
                  axis=0, keepdims=True)
    bt = batch_ref[...]
    iot = lax.broadcasted_iota(jnp.int32, (1, _B), 1)
    cnt = jnp.sum((bt == iot).astype(jnp.float32), axis=0,
                  keepdims=True)
    pooled = num / jnp.maximum(cnt, 1.0)
    out_ref[...] = jax.nn.sigmoid(pooled + bout_ref[0, 0])


def _pool_tail(acc, batch2d, bout2d):
    return pl.pallas_call(
        _out_body,
        out_shape=jax.ShapeDtypeStruct((1, _B), jnp.float32),
    )(acc, batch2d, bout2d)


def kernel(x, edge_index, batch, W1, W2, W_out, b_out):
    n, _ = x.shape
    h_dim = W1.shape[1]
    e = edge_index.shape[1]

    src = edge_index[0].astype(jnp.int32)
    dst = edge_index[1].astype(jnp.int32)
    batch_i = batch.astype(jnp.int32)

    nw = _NC * _NS
    ept = e // nw
    nch = ept // _CH
    main = nch * _CH
    dst2 = dst.reshape(nw, ept)
    dst3 = dst2[:, :main].reshape(nw, nch, _CH)
    dstt = dst2[:, main:]

    h = _h_matmul(x, W1)
    zeros = jnp.zeros((1000, h_dim), jnp.float32)
    g_part = _make_sc_msg_pass(n, h_dim, e)(src, dst3, dstt, h, zeros)
    s = _s_matvec(g_part, W2, W_out)
    acc = _make_sc_scalar_pass(n, e)(src, dst, s, batch_i)
    out = _pool_tail(acc, batch_i.reshape(n, 1), b_out.reshape(1, 1))
    return out.reshape(_B, 1)


# speedup vs baseline: 1.3202x; 1.0124x over previous
import functools

import jax
import jax.numpy as jnp
from jax import lax
from jax.experimental import pallas as pl
from jax.experimental.pallas import tpu as pltpu
from jax.experimental.pallas import tpu_sc as plsc

_B = 64
_NC = 2
_NS = 16
_LANES = 16


def _h_body(x_ref, w1_ref, h_ref):
    h_ref[...] = jnp.dot(x_ref[...], w1_ref[...],
                         preferred_element_type=jnp.float32)


def _h_matmul(x, w1):
    n, _ = x.shape
    h = w1.shape[1]
    return pl.pallas_call(
        _h_body,
        out_shape=jax.ShapeDtypeStruct((n, h), jnp.float32),
    )(x, w1)


_CH = 96


def _make_sc_msg_pass(n, h, e):
    nw = _NC * _NS
    ept = e // nw
    nch = ept // _CH
    tail = ept - nch * _CH
    rp = 1000
    nt = n // rp
    assert ept * nw == e and nch % 2 == 0 and 0 < tail <= _CH
    assert rp % 8 == 0 and rp * nt == n and nt <= _NS and tail % 8 == 0

    mesh = plsc.VectorSubcoreMesh(core_axis_name="c", subcore_axis_name="s")

    @functools.partial(
        pl.kernel,
        out_type=jax.ShapeDtypeStruct((_NC, n, h), jnp.float32),
        mesh=mesh,
        scratch_types=[
            pltpu.VMEM((ept,), jnp.int32),
            pltpu.VMEM((nch, _CH), jnp.int32),
            pltpu.VMEM((tail,), jnp.int32),
            pltpu.VMEM((_CH, h), jnp.float32),
            pltpu.VMEM((_CH, h), jnp.float32),
            pltpu.VMEM_SHARED((n, h), jnp.float32),
            pltpu.SemaphoreType.DMA,
            pltpu.SemaphoreType.DMA,
        ],
    )
    def sc_msg_pass(src_hbm, dst3_hbm, dstt_hbm, h_hbm, zeros_hbm, out_hbm,
                    src_v, dst_v, dstt_v, rows0, rows1, acc_sh, sem0, sem1):
        cid = lax.axis_index("c")
        sid = lax.axis_index("s")
        wid = cid * _NS + sid

        @pl.when(sid < nt)
        def _():
            pltpu.sync_copy(zeros_hbm, acc_sh.at[pl.ds(sid * rp, rp)])

        pltpu.sync_copy(src_hbm.at[pl.ds(wid * ept, ept)], src_v)
        pltpu.sync_copy(dst3_hbm.at[wid], dst_v)
        pltpu.sync_copy(dstt_hbm.at[wid], dstt_v)
        plsc.subcore_barrier()

        def sidx(c):
            return src_v.at[pl.ds(c * _CH, _CH)]

        def didx(c):
            return dst_v.at[c]

        pltpu.async_copy(h_hbm.at[sidx(0)], rows0, sem0)
        pltpu.async_copy(h_hbm.at[sidx(1)], rows1, sem1)

        def body(g, carry):
            a = 2 * g
            pltpu.make_async_copy(h_hbm.at[sidx(a)], rows0, sem0).wait()
            pltpu.sync_copy(rows0, acc_sh.at[didx(a)], add=True)

            @pl.when(a + 2 < nch)
            def _():
                pltpu.async_copy(h_hbm.at[sidx(a + 2)], rows0, sem0)

            pltpu.make_async_copy(h_hbm.at[sidx(a + 1)], rows1, sem1).wait()
            pltpu.sync_copy(rows1, acc_sh.at[didx(a + 1)], add=True)

            @pl.when(a + 3 < nch)
            def _():
                pltpu.async_copy(h_hbm.at[sidx(a + 3)], rows1, sem1)

            return carry

        lax.fori_loop(0, nch // 2, body, 0)

        rowst = rows0.at[pl.ds(0, tail)]
        pltpu.async_copy(h_hbm.at[src_v.at[pl.ds(nch * _CH, tail)]],
                         rowst, sem0).wait()
        pltpu.sync_copy(rowst, acc_sh.at[dstt_v], add=True)
        plsc.subcore_barrier()

        @pl.when(sid < nt)
        def _():
            pltpu.sync_copy(acc_sh.at[pl.ds(sid * rp, rp)],
                            out_hbm.at[cid, pl.ds(sid * rp, rp)])

    return sc_msg_pass


def _s_body(g_ref, w2_ref, wout_ref, s_ref):
    g = jnp.maximum(g_ref[0] + g_ref[1], 0.0)
    v = jnp.dot(w2_ref[...], wout_ref[...],
                preferred_element_type=jnp.float32)
    s_ref[...] = jnp.dot(g, v, preferred_element_type=jnp.float32)[:, 0]


def _s_matvec(g_part, w2, w_out):
    n = g_part.shape[1]
    return pl.pallas_call(
        _s_body,
        out_shape=jax.ShapeDtypeStruct((n,), jnp.float32),
    )(g_part, w2, w_out)


def _make_sc_scalar_pass(n, e):
    nw = _NC * _NS
    ept = e // nw
    nvec = ept // _LANES
    assert ept * nw == e and nvec * _LANES == ept

    mesh = plsc.VectorSubcoreMesh(core_axis_name="c", subcore_axis_name="s")

    @functools.partial(
        pl.kernel,
        out_type=jax.ShapeDtypeStruct((nw, _LANES, _B), jnp.float32),
        mesh=mesh,
        scratch_types=[
            pltpu.VMEM((ept,), jnp.int32),
            pltpu.VMEM((ept,), jnp.int32),
            pltpu.VMEM((n,), jnp.float32),
            pltpu.VMEM((n,), jnp.int32),
            pltpu.VMEM((_LANES, _B), jnp.float32),
            pltpu.SemaphoreType.DMA,
        ],
        compiler_params=pltpu.CompilerParams(needs_layout_passes=False),
    )
    def sc_scalar_pass(src_hbm, dst_hbm, s_hbm, batch_hbm, out_hbm,
                       src_v, dst_v, s_v, b_v, acc_v, sem):
        cid = lax.axis_index("c")
        sid = lax.axis_index("s")
        wid = cid * _NS + sid
        pltpu.async_copy(s_hbm, s_v, sem)
        pltpu.async_copy(batch_hbm, b_v, sem)
        pltpu.async_copy(src_hbm.at[pl.ds(wid * ept, ept)], src_v, sem)
        pltpu.async_copy(dst_hbm.at[pl.ds(wid * ept, ept)], dst_v, sem)

        zero = jnp.zeros((_LANES,), jnp.float32)
        for r in range(_LANES):
            for c0 in range(_B // _LANES):
                acc_v[r, pl.ds(c0 * _LANES, _LANES)] = zero

        pltpu.make_async_copy(s_hbm, s_v, sem).wait()
        pltpu.make_async_copy(batch_hbm, b_v, sem).wait()
        pltpu.make_async_copy(src_hbm.at[pl.ds(0, ept)], src_v, sem).wait()
        pltpu.make_async_copy(dst_hbm.at[pl.ds(0, ept)], dst_v, sem).wait()

        lane = lax.iota(jnp.int32, _LANES)
        unroll = 5
        assert nvec % unroll == 0

        def body(i, carry):
            base = i * (unroll * _LANES)
            for u in range(unroll):
                sl = pl.ds(base + u * _LANES, _LANES)
                src16 = src_v[sl]
                dst16 = dst_v[sl]
                sval = plsc.load_gather(s_v, [src16])
                bval = plsc.load_gather(b_v, [dst16])
                plsc.addupdate_scatter(acc_v, [lane, bval], sval)
            return carry

        lax.fori_loop(0, nvec // unroll, body, 0)
        pltpu.sync_copy(acc_v, out_hbm.at[wid])

    return sc_scalar_pass


def _out_body(acc_ref, batch_ref, bout_ref, out_ref):
    acc = acc_ref[...]
    num = jnp.sum(acc.reshape(acc.shape[0] * acc.shape[1], _B),
                  axis=0, keepdims=True)
    bt = batch_ref[...]
    iot = lax.broadcasted_iota(jnp.int32, (1, _B), 1)
    cnt = jnp.sum((bt == iot).astype(jnp.float32), axis=0,
                  keepdims=True)
    pooled = num / jnp.maximum(cnt, 1.0)
    out_ref[...] = jax.nn.sigmoid(pooled + bout_ref[0, 0])


def _pool_tail(acc, batch2d, bout2d):
    return pl.pallas_call(
        _out_body,
        out_shape=jax.ShapeDtypeStruct((1, _B), jnp.float32),
    )(acc, batch2d, bout2d)


def kernel(x, edge_index, batch, W1, W2, W_out, b_out):
    n, _ = x.shape
    h_dim = W1.shape[1]
    e = edge_index.shape[1]

    src = edge_index[0].astype(jnp.int32)
    dst = edge_index[1].astype(jnp.int32)
    batch_i = batch.astype(jnp.int32)

    nw = _NC * _NS
    ept = e // nw
    nch = ept // _CH
    main = nch * _CH
    dst2 = dst.reshape(nw, ept)
    dst3 = dst2[:, :main].reshape(nw, nch, _CH)
    dstt = dst2[:, main:]

    h = _h_matmul(x, W1)
    zeros = jnp.zeros((1000, h_dim), jnp.float32)
    g_part = _make_sc_msg_pass(n, h_dim, e)(src, dst3, dstt, h, zeros)
    s = _s_matvec(g_part, W2, W_out)
    acc = _make_sc_scalar_pass(n, e)(src, dst, s, batch_i)
    out = _pool_tail(acc, batch_i.reshape(n, 1), b_out.reshape(1, 1))
    return out.reshape(_B, 1)


# speedup vs baseline: 1.3219x; 1.0012x over previous
import functools

import jax
import jax.numpy as jnp
from jax import lax
from jax.experimental import pallas as pl
from jax.experimental.pallas import tpu as pltpu
from jax.experimental.pallas import tpu_sc as plsc

_B = 64
_NC = 2
_NS = 16
_LANES = 16


def _h_body(x_ref, w1_ref, h_ref):
    h_ref[...] = jnp.dot(x_ref[...], w1_ref[...],
                         preferred_element_type=jnp.float32)


def _h_matmul(x, w1):
    n, _ = x.shape
    h = w1.shape[1]
    return pl.pallas_call(
        _h_body,
        out_shape=jax.ShapeDtypeStruct((n, h), jnp.float32),
    )(x, w1)


_CH = 96


def _make_sc_msg_pass(n, h, e):
    nw = _NC * _NS
    ept = e // nw
    nch = ept // _CH
    tail = ept - nch * _CH
    rp = 1000
    nt = n // rp
    assert ept * nw == e and nch % 2 == 0 and 0 < tail <= _CH
    assert rp % 8 == 0 and rp * nt == n and nt <= _NS and tail % 8 == 0

    mesh = plsc.VectorSubcoreMesh(core_axis_name="c", subcore_axis_name="s")

    @functools.partial(
        pl.kernel,
        out_type=jax.ShapeDtypeStruct((_NC, n, h), jnp.float32),
        mesh=mesh,
        scratch_types=[
            pltpu.VMEM((ept,), jnp.int32),
            pltpu.VMEM((nch, _CH), jnp.int32),
            pltpu.VMEM((tail,), jnp.int32),
            pltpu.VMEM((_CH, h), jnp.float32),
            pltpu.VMEM((_CH, h), jnp.float32),
            pltpu.VMEM_SHARED((n, h), jnp.float32),
            pltpu.SemaphoreType.DMA,
            pltpu.SemaphoreType.DMA,
        ],
    )
    def sc_msg_pass(src_hbm, dst_hbm, h_hbm, zeros_hbm, out_hbm,
                    src_v, dst_v, dstt_v, rows0, rows1, acc_sh, sem0, sem1):
        cid = lax.axis_index("c")
        sid = lax.axis_index("s")
        wid = cid * _NS + sid

        @pl.when(sid < nt)
        def _():
            pltpu.sync_copy(zeros_hbm, acc_sh.at[pl.ds(sid * rp, rp)])

        pltpu.sync_copy(src_hbm.at[pl.ds(wid * ept, ept)], src_v)
        base = wid * ept
        grp = 8
        assert nch % grp == 0

        def dst_prefetch(i, carry):
            for j in range(grp):
                c = i * grp + j
                pltpu.async_copy(dst_hbm.at[pl.ds(base + c * _CH, _CH)],
                                 dst_v.at[c], sem1)
            for j in range(grp):
                c = i * grp + j
                pltpu.make_async_copy(dst_hbm.at[pl.ds(base, _CH)],
                                      dst_v.at[c], sem1).wait()
            return carry

        lax.fori_loop(0, nch // grp, dst_prefetch, 0)
        pltpu.sync_copy(dst_hbm.at[pl.ds(base + nch * _CH, tail)], dstt_v)
        plsc.subcore_barrier()

        def sidx(c):
            return src_v.at[pl.ds(c * _CH, _CH)]

        def didx(c):
            return dst_v.at[c]

        pltpu.async_copy(h_hbm.at[sidx(0)], rows0, sem0)
        pltpu.async_copy(h_hbm.at[sidx(1)], rows1, sem1)

        def body(g, carry):
            a = 2 * g
            pltpu.make_async_copy(h_hbm.at[sidx(a)], rows0, sem0).wait()
            pltpu.sync_copy(rows0, acc_sh.at[didx(a)], add=True)

            @pl.when(a + 2 < nch)
            def _():
                pltpu.async_copy(h_hbm.at[sidx(a + 2)], rows0, sem0)

            pltpu.make_async_copy(h_hbm.at[sidx(a + 1)], rows1, sem1).wait()
            pltpu.sync_copy(rows1, acc_sh.at[didx(a + 1)], add=True)

            @pl.when(a + 3 < nch)
            def _():
                pltpu.async_copy(h_hbm.at[sidx(a + 3)], rows1, sem1)

            return carry

        lax.fori_loop(0, nch // 2, body, 0)

        rowst = rows0.at[pl.ds(0, tail)]
        pltpu.async_copy(h_hbm.at[src_v.at[pl.ds(nch * _CH, tail)]],
                         rowst, sem0).wait()
        pltpu.sync_copy(rowst, acc_sh.at[dstt_v], add=True)
        plsc.subcore_barrier()

        @pl.when(sid < nt)
        def _():
            pltpu.sync_copy(acc_sh.at[pl.ds(sid * rp, rp)],
                            out_hbm.at[cid, pl.ds(sid * rp, rp)])

    return sc_msg_pass


def _s_body(g_ref, w2_ref, wout_ref, s_ref):
    g = jnp.maximum(g_ref[0] + g_ref[1], 0.0)
    v = jnp.dot(w2_ref[...], wout_ref[...],
                preferred_element_type=jnp.float32)
    s_ref[...] = jnp.dot(g, v, preferred_element_type=jnp.float32)[:, 0]


def _s_matvec(g_part, w2, w_out):
    n = g_part.shape[1]
    return pl.pallas_call(
        _s_body,
        out_shape=jax.ShapeDtypeStruct((n,), jnp.float32),
    )(g_part, w2, w_out)


def _make_sc_scalar_pass(n, e):
    nw = _NC * _NS
    ept = e // nw
    nvec = ept // _LANES
    assert ept * nw == e and nvec * _LANES == ept

    mesh = plsc.VectorSubcoreMesh(core_axis_name="c", subcore_axis_name="s")

    @functools.partial(
        pl.kernel,
        out_type=jax.ShapeDtypeStruct((nw, _LANES, _B), jnp.float32),
        mesh=mesh,
        scratch_types=[
            pltpu.VMEM((ept,), jnp.int32),
            pltpu.VMEM((ept,), jnp.int32),
            pltpu.VMEM((n,), jnp.float32),
            pltpu.VMEM((n,), jnp.int32),
            pltpu.VMEM((_LANES, _B), jnp.float32),
            pltpu.SemaphoreType.DMA,
        ],
        compiler_params=pltpu.CompilerParams(needs_layout_passes=False),
    )
    def sc_scalar_pass(src_hbm, dst_hbm, s_hbm, batch_hbm, out_hbm,
                       src_v, dst_v, s_v, b_v, acc_v, sem):
        cid = lax.axis_index("c")
        sid = lax.axis_index("s")
        wid = cid * _NS + sid
        pltpu.async_copy(s_hbm, s_v, sem)
        pltpu.async_copy(batch_hbm, b_v, sem)
        pltpu.async_copy(src_hbm.at[pl.ds(wid * ept, ept)], src_v, sem)
        pltpu.async_copy(dst_hbm.at[pl.ds(wid * ept, ept)], dst_v, sem)

        zero = jnp.zeros((_LANES,), jnp.float32)
        for r in range(_LANES):
            for c0 in range(_B // _LANES):
                acc_v[r, pl.ds(c0 * _LANES, _LANES)] = zero

        pltpu.make_async_copy(s_hbm, s_v, sem).wait()
        pltpu.make_async_copy(batch_hbm, b_v, sem).wait()
        pltpu.make_async_copy(src_hbm.at[pl.ds(0, ept)], src_v, sem).wait()
        pltpu.make_async_copy(dst_hbm.at[pl.ds(0, ept)], dst_v, sem).wait()

        lane = lax.iota(jnp.int32, _LANES)
        unroll = 5
        assert nvec % unroll == 0

        def body(i, carry):
            base = i * (unroll * _LANES)
            for u in range(unroll):
                sl = pl.ds(base + u * _LANES, _LANES)
                src16 = src_v[sl]
                dst16 = dst_v[sl]
                sval = plsc.load_gather(s_v, [src16])
                bval = plsc.load_gather(b_v, [dst16])
                plsc.addupdate_scatter(acc_v, [lane, bval], sval)
            return carry

        lax.fori_loop(0, nvec // unroll, body, 0)
        pltpu.sync_copy(acc_v, out_hbm.at[wid])

    return sc_scalar_pass


def _out_body(acc_ref, batch_ref, bout_ref, out_ref):
    acc = acc_ref[...]
    num = jnp.sum(acc.reshape(acc.shape[0] * acc.shape[1], _B),
                  axis=0, keepdims=True)
    bt = batch_ref[...]
    iot = lax.broadcasted_iota(jnp.int32, (1, _B), 1)
    cnt = jnp.sum((bt == iot).astype(jnp.float32), axis=0,
                  keepdims=True)
    pooled = num / jnp.maximum(cnt, 1.0)
    out_ref[...] = jax.nn.sigmoid(pooled + bout_ref[0, 0])


def _pool_tail(acc, batch2d, bout2d):
    return pl.pallas_call(
        _out_body,
        out_shape=jax.ShapeDtypeStruct((1, _B), jnp.float32),
    )(acc, batch2d, bout2d)


def kernel(x, edge_index, batch, W1, W2, W_out, b_out):
    n, _ = x.shape
    h_dim = W1.shape[1]
    e = edge_index.shape[1]

    src = edge_index[0].astype(jnp.int32)
    dst = edge_index[1].astype(jnp.int32)
    batch_i = batch.astype(jnp.int32)

    h = _h_matmul(x, W1)
    zeros = jnp.zeros((1000, h_dim), jnp.float32)
    g_part = _make_sc_msg_pass(n, h_dim, e)(src, dst, h, zeros)
    s = _s_matvec(g_part, W2, W_out)
    acc = _make_sc_scalar_pass(n, e)(src, dst, s, batch_i)
    out = _pool_tail(acc, batch_i.reshape(n, 1), b_out.reshape(1, 1))
    return out.reshape(_B, 1)
